# baseline (speedup 1.0000x reference)
import functools

import jax
import jax.numpy as jnp
from jax import lax
from jax.experimental import pallas as pl
from jax.experimental.pallas import tpu as pltpu
from jax.experimental.pallas import tpu_sc as plsc


def _build_sc_embed(R, D, n_workers, P):
    per_w = R // n_workers
    rows = P // 128
    n_chunks = per_w // P

    mesh = plsc.VectorSubcoreMesh(core_axis_name="c", subcore_axis_name="s")
    nc = 2

    @functools.partial(
        pl.kernel,
        mesh=mesh,
        out_type=jax.ShapeDtypeStruct((R, D), jnp.float32),
        compiler_params=pltpu.CompilerParams(
            use_tc_tiling_on_sc=False, needs_layout_passes=False),
        scratch_types=[
            pltpu.VMEM((2 * P,), jnp.int32),
            pltpu.VMEM((2 * rows, 128), jnp.int32),
            pltpu.VMEM((P, D), jnp.float32),
            pltpu.SemaphoreType.DMA,
        ],
    )
    def body(xi_hbm, e_hbm, out_hbm, il, idx_v, buf, sem):
        wid = lax.axis_index("s") * nc + lax.axis_index("c")
        two_iota = lax.iota(jnp.int32, 16) * 2

        def do_chunk(c, carry):
            base = wid * per_w + c * P
            pltpu.sync_copy(xi_hbm.at[pl.ds(2 * base, 2 * P)], il)
            for r in range(rows):
                for g in range(8):
                    src = two_iota + (256 * r + 32 * g)
                    idx_v[r, pl.ds(16 * g, 16)] = plsc.load_gather(il, [src])
                    idx_v[rows + r, pl.ds(16 * g, 16)] = plsc.load_gather(
                        il, [src + 1])
            cps = [
                pltpu.async_copy(
                    e_hbm.at[idx_v.at[r]], buf.at[pl.ds(r * 128, 128)], sem)
                for r in range(rows)
            ]
            for cp in cps:
                cp.wait()
            cps = [
                pltpu.async_copy(
                    e_hbm.at[idx_v.at[rows + r]], buf.at[pl.ds(r * 128, 128)],
                    sem, add=True)
                for r in range(rows)
            ]
            for cp in cps:
                cp.wait()
            pltpu.sync_copy(buf, out_hbm.at[pl.ds(base, P)])
            return carry

        lax.fori_loop(0, n_chunks, do_chunk, 0)

    return body


def kernel(x, E):
    B, L, H = x.shape
    V, D = E.shape
    assert H == 2 and D % 16 == 0
    R = B * L
    n_workers = 32
    P = 512
    assert R % (n_workers * P) == 0 and P % 128 == 0

    xi = x.astype(jnp.int32).reshape(R * 2)
    body = _build_sc_embed(R, D, n_workers, P)
    out = body(xi, E)
    return out.reshape(B, L, D)


# speedup vs baseline: 1.8090x; 1.8090x over previous
import functools

import jax
import jax.numpy as jnp
from jax import lax
from jax.experimental import pallas as pl
from jax.experimental.pallas import tpu as pltpu
from jax.experimental.pallas import tpu_sc as plsc

_NW = 32
_KB = 4


def _build_sc_embed(B, L, V, D):
    NB = B // 128
    n_units = L * NB
    su_per_w = n_units // (_NW * _KB)
    su_per_l = NB // _KB

    mesh = plsc.VectorSubcoreMesh(core_axis_name="c", subcore_axis_name="s")

    @functools.partial(
        pl.kernel,
        mesh=mesh,
        out_type=jax.ShapeDtypeStruct((L, D // 8, NB, 8, 128), jnp.float32),
        compiler_params=pltpu.CompilerParams(
            use_tc_tiling_on_sc=False, needs_layout_passes=False),
        scratch_types=[
            pltpu.VMEM((_KB, 2, 128), jnp.int32),
            pltpu.VMEM((_KB * 128, D), jnp.float32),
            pltpu.VMEM((_KB, D, 128), jnp.float32),
            pltpu.SemaphoreType.DMA,
            pltpu.SemaphoreType.DMA,
            pltpu.SemaphoreType.DMA,
            pltpu.SemaphoreType.DMA,
            pltpu.SemaphoreType.DMA,
        ],
    )
    def body(xl_hbm, e_hbm, ol_hbm, idxv, buf, obuf, s0, s1, s2, s3, so):
        wid = lax.axis_index("s") * 2 + lax.axis_index("c")
        sems = [s0, s1, s2, s3]
        iota = lax.iota(jnp.int32, 16)
        iota_hi = iota + 16

        def do_su(s, carry):
            su = wid * su_per_w + s
            l = su // su_per_l
            bt0 = (su % su_per_l) * _KB
            pltpu.sync_copy(xl_hbm.at[l, pl.ds(bt0, _KB)], idxv)
            g0 = [
                pltpu.async_copy(
                    e_hbm.at[idxv.at[k, 0]],
                    buf.at[pl.ds(k * 128, 128)], sems[k])
                for k in range(_KB)
            ]
            g1 = []
            for k in range(_KB):
                g0[k].wait()
                g1.append(pltpu.async_copy(
                    e_hbm.at[idxv.at[k, 1]],
                    buf.at[pl.ds(k * 128, 128)], sems[k], add=True))
            for k in range(_KB):
                g1[k].wait()

            outs = []
            for k in range(_KB):
                obk = obuf.at[k]

                def tr_body(i2, carry2, _k=k, _obk=obk):
                    for u in range(4):
                        i = i2 * 4 + u
                        col = jnp.full((16,), i, jnp.int32)
                        plsc.store_scatter(
                            _obk, [iota, col],
                            buf[_k * 128 + i, pl.ds(0, 16)])
                        plsc.store_scatter(
                            _obk, [iota_hi, col],
                            buf[_k * 128 + i, pl.ds(16, 16)])
                    return carry2

                lax.fori_loop(0, 32, tr_body, 0)
                for dt in range(D // 8):
                    outs.append(pltpu.async_copy(
                        obuf.at[k, pl.ds(8 * dt, 8)],
                        ol_hbm.at[l, dt, bt0 + k], so))
            for cp in outs:
                cp.wait()
            return carry

        lax.fori_loop(0, su_per_w, do_su, 0)

    return body


def kernel(x, E):
    B, L, H = x.shape
    V, D = E.shape
    assert H == 2 and D % 16 == 0 and B % 128 == 0
    xl = (x.astype(jnp.int32)
          .transpose(1, 2, 0)
          .reshape(L, H, B // 128, 128)
          .transpose(0, 2, 1, 3))
    body = _build_sc_embed(B, L, V, D)
    ol = body(xl, E)
    return ol.transpose(2, 4, 0, 1, 3).reshape(B, L, D)


# speedup vs baseline: 1.8587x; 1.0275x over previous
import functools

import jax
import jax.numpy as jnp
from jax import lax
from jax.experimental import pallas as pl
from jax.experimental.pallas import tpu as pltpu
from jax.experimental.pallas import tpu_sc as plsc

_NW = 32
_KB = 4


def _build_sc_embed(B, L, V, D):
    NB = B // 128
    ND = D // 8
    n_units = L * NB
    S = n_units // (_NW * _KB)
    su_per_l = NB // _KB
    assert S % 2 == 0

    mesh = plsc.VectorSubcoreMesh(core_axis_name="c", subcore_axis_name="s")

    @functools.partial(
        pl.kernel,
        mesh=mesh,
        out_type=jax.ShapeDtypeStruct((L, ND, NB, 8, 128), jnp.float32),
        compiler_params=pltpu.CompilerParams(
            use_tc_tiling_on_sc=False, needs_layout_passes=False),
        scratch_types=[
            pltpu.VMEM((2, _KB, 2, 128), jnp.int32),
            pltpu.VMEM((2, _KB * 128, D), jnp.float32),
            pltpu.VMEM((2, ND, _KB, 8, 128), jnp.float32),
        ]
        + [pltpu.SemaphoreType.DMA] * (2 * _KB)
        + [pltpu.SemaphoreType.DMA] * 2
        + [pltpu.SemaphoreType.DMA] * 2,
    )
    def body(xl_hbm, e_hbm, ol_hbm, idxv, buf, obuf, *sems):
        sg = [sems[:_KB], sems[_KB:2 * _KB]]
        si = sems[2 * _KB:2 * _KB + 2]
        so = sems[2 * _KB + 2:]
        wid = lax.axis_index("s") * 2 + lax.axis_index("c")
        iota = lax.iota(jnp.int32, 16)
        dt_lo = lax.shift_right_logical(iota, 3)
        dt_hi = dt_lo + 2
        r_vec = lax.bitwise_and(iota, 7)

        def unit_pos(su):
            return su // su_per_l, (su % su_per_l) * _KB

        def fire_g0(b, k):
            return pltpu.async_copy(
                e_hbm.at[idxv.at[b, k, 0]],
                buf.at[b, pl.ds(k * 128, 128)], sg[b][k])

        def fire_idx(b, su):
            l, bt0 = unit_pos(su)
            return pltpu.async_copy(
                xl_hbm.at[l, pl.ds(bt0, _KB)], idxv.at[b], si[b])

        def process(s, b):
            su = wid * S + s
            l, bt0 = unit_pos(su)
            @pl.when(s < S - 1)
            def _():
                pltpu.make_async_copy(
                    xl_hbm.at[0, pl.ds(0, _KB)], idxv.at[1 - b],
                    si[1 - b]).wait()
                for k in range(_KB):
                    fire_g0(1 - b, k)
            for k in range(_KB):
                pltpu.make_async_copy(
                    e_hbm.at[idxv.at[b, k, 0]],
                    buf.at[b, pl.ds(k * 128, 128)], sg[b][k]).wait()
                pltpu.async_copy(
                    e_hbm.at[idxv.at[b, k, 1]],
                    buf.at[b, pl.ds(k * 128, 128)], sg[b][k], add=True)
            @pl.when(s >= 2)
            def _():
                for dt in range(ND):
                    pltpu.make_async_copy(
                        obuf.at[b, dt],
                        ol_hbm.at[l, dt, pl.ds(bt0, _KB)], so[b]).wait()
            for k in range(_KB):
                pltpu.make_async_copy(
                    e_hbm.at[idxv.at[b, k, 1]],
                    buf.at[b, pl.ds(k * 128, 128)], sg[b][k]).wait()
                kvec = jnp.full((16,), k, jnp.int32)

                def tr_body(i2, carry2, _k=k, _kvec=kvec):
                    for u in range(4):
                        i = i2 * 4 + u
                        col = jnp.full((16,), i, jnp.int32)
                        plsc.store_scatter(
                            obuf.at[b], [dt_lo, _kvec, r_vec, col],
                            buf[b, _k * 128 + i, pl.ds(0, 16)])
                        plsc.store_scatter(
                            obuf.at[b], [dt_hi, _kvec, r_vec, col],
                            buf[b, _k * 128 + i, pl.ds(16, 16)])
                    return carry2

                lax.fori_loop(0, 32, tr_body, 0)
            @pl.when(s < S - 2)
            def _():
                fire_idx(b, su + 2)
            for dt in range(ND):
                pltpu.async_copy(
                    obuf.at[b, dt], ol_hbm.at[l, dt, pl.ds(bt0, _KB)], so[b])

        su0 = wid * S
        l0, b00 = unit_pos(su0)
        pltpu.sync_copy(xl_hbm.at[l0, pl.ds(b00, _KB)], idxv.at[0])
        for k in range(_KB):
            fire_g0(0, k)
        fire_idx(1, su0 + 1)

        def do_pair(i, carry):
            process(2 * i, 0)
            process(2 * i + 1, 1)
            return carry

        lax.fori_loop(0, S // 2, do_pair, 0)
        lf, btf = unit_pos(su0)
        for b in range(2):
            for dt in range(ND):
                pltpu.make_async_copy(
                    obuf.at[b, dt],
                    ol_hbm.at[lf, dt, pl.ds(btf, _KB)], so[b]).wait()

    return body


def kernel(x, E):
    B, L, H = x.shape
    V, D = E.shape
    assert H == 2 and D % 16 == 0 and B % 128 == 0
    xl = (x.astype(jnp.int32)
          .transpose(1, 2, 0)
          .reshape(L, H, B // 128, 128)
          .transpose(0, 2, 1, 3))
    body = _build_sc_embed(B, L, V, D)
    ol = body(xl, E)
    return ol.transpose(2, 4, 0, 1, 3).reshape(B, L, D)
